# baseline (speedup 1.0000x reference)
import functools

import jax
import numpy as np
import jax.numpy as jnp
from jax import lax
from jax.experimental import pallas as pl
from jax.experimental.pallas import tpu as pltpu
from jax.experimental.pallas import tpu_sc as plsc

N = 10000
E = 160000
AUX = 16
OUT = 256
HALF = OUT // 2
NEG = 0.2

NC, NS, L = 2, 16, 16
NW = NC * NS
CH = 128
CC = 64
BLK = 512
SUB = BLK // CH
E_PAD = 163840
EPW = E_PAD // NW
EPT = E_PAD // NS
NB1 = EPT // BLK
NB2 = EPW // BLK
NCC = EPT // CC
PACK = 16384
N_PAD = 10112
ROWS_PT = N_PAD // NS

_MESH = plsc.VectorSubcoreMesh(core_axis_name="c", subcore_axis_name="s")
_f32 = jnp.float32
_bf16 = jnp.bfloat16
_NOTC = pltpu.CompilerParams(use_tc_tiling_on_sc=False,
                             needs_layout_passes=False)

def _mk_perm():
    p = np.zeros(OUT, dtype=np.int32)
    for q in range(OUT // 32):
        for m in range(16):
            p[32 * q + 2 * m] = 32 * q + m
            p[32 * q + 2 * m + 1] = 32 * q + 16 + m
    return p

_P256 = _mk_perm()


def _leaky_exp(x):
    return jnp.exp(jnp.where(x >= 0, x, NEG * x))


def _vloop(n, body, unroll=8):
    def f(j, carry):
        body(j)
        return carry
    lax.fori_loop(0, n, f, 0, unroll=unroll)


@functools.partial(
    pl.kernel,
    out_type=jax.ShapeDtypeStruct((E_PAD, AUX), _f32),
    mesh=_MESH,
    compiler_params=_NOTC,
    scratch_types=[
        pltpu.VMEM_SHARED((N_PAD, AUX), _f32),
        pltpu.VMEM((BLK, AUX), _f32),
        pltpu.VMEM((BLK, AUX), _f32),
        pltpu.VMEM((BLK, AUX), _f32),
        pltpu.VMEM((BLK, AUX), _f32),
        pltpu.VMEM((NB1 * SUB, CH), jnp.int32),
        pltpu.VMEM((NB2 * SUB, CH), jnp.int32),
        pltpu.SemaphoreType.DMA,
        pltpu.SemaphoreType.DMA,
        pltpu.SemaphoreType.DMA,
        pltpu.SemaphoreType.DMA,
    ],
)
def _stage_a(p_hbm, dst2d_hbm, zero_hbm, a_hbm,
             den_sh, p0, p1, d0, d1, dstA, dstB,
             semp0, semp1, semg0, semg1):
    c = lax.axis_index("c")
    s = lax.axis_index("s")
    wid = s * NC + c
    row0 = s * ROWS_PT
    pltpu.sync_copy(zero_hbm.at[pl.ds(row0, ROWS_PT)],
                    den_sh.at[pl.ds(row0, ROWS_PT)])
    pltpu.sync_copy(dst2d_hbm.at[pl.ds(s * NB1 * SUB, NB1 * SUB)], dstA)
    pltpu.sync_copy(dst2d_hbm.at[pl.ds(wid * NB2 * SUB, NB2 * SUB)], dstB)
    plsc.subcore_barrier()

    pbufs = (p0, p1)
    dbufs = (d0, d1)
    psems = (semp0, semp1)
    gsems = (semg0, semg1)

    def p_src1(i):
        return p_hbm.at[pl.ds((s * NB1 + i) * BLK, BLK)]

    pltpu.async_copy(p_src1(0), p0, semp0)
    pltpu.async_copy(p_src1(1), p1, semp1)

    def blk1(k, _):
        for b in (0, 1):
            i = 2 * k + b
            pb = pbufs[b]
            pltpu.make_async_copy(p_src1(i), pb, psems[b]).wait()

            def row(j):
                pb[j, :] = _leaky_exp(pb[j, :])

            _vloop(BLK, row)
            for u in range(SUB):
                pltpu.sync_copy(pb.at[pl.ds(u * CH, CH)],
                                den_sh.at[dstA.at[i * SUB + u]], add=True)

            @pl.when(i + 2 < NB1)
            def _():
                pltpu.async_copy(p_src1(i + 2), pb, psems[b])
        return 0

    lax.fori_loop(0, NB1 // 2, blk1, 0, unroll=False)
    plsc.subcore_barrier()

    def p_src2(i):
        return p_hbm.at[pl.ds((wid * NB2 + i) * BLK, BLK)]

    def gathers(i, b):
        for u in range(SUB):
            pltpu.async_copy(den_sh.at[dstB.at[i * SUB + u]],
                             dbufs[b].at[pl.ds(u * CH, CH)], gsems[b])

    pltpu.async_copy(p_src2(0), p0, semp0)
    gathers(0, 0)
    pltpu.async_copy(p_src2(1), p1, semp1)
    gathers(1, 1)

    def blk2(k, _):
        for b in (0, 1):
            i = 2 * k + b
            pb = pbufs[b]
            db = dbufs[b]
            pltpu.make_async_copy(p_src2(i), pb, psems[b]).wait()
            for u in range(SUB):
                pltpu.make_async_copy(den_sh.at[dstB.at[i * SUB + u]],
                                      db.at[pl.ds(u * CH, CH)],
                                      gsems[b]).wait()

            def row(j):
                pb[j, :] = _leaky_exp(pb[j, :]) / db[j, :]

            _vloop(BLK, row)
            pltpu.sync_copy(pb, a_hbm.at[pl.ds((wid * NB2 + i) * BLK, BLK)])

            @pl.when(i + 2 < NB2)
            def _():
                pltpu.async_copy(p_src2(i + 2), pb, psems[b])
                gathers(i + 2, b)
        return 0

    lax.fori_loop(0, NB2 // 2, blk2, 0, unroll=False)


def _mm_body(a_ref, wt_ref, b_ref, lo_ref, hi_ref):
    r = jnp.dot(a_ref[...], wt_ref[...], preferred_element_type=_f32)
    r = (r + b_ref[...]).astype(_bf16)
    lo_ref[...] = r[:, :HALF]
    hi_ref[...] = r[:, HALF:]


def _linear(a, wt, b):
    MB = 1024
    return pl.pallas_call(
        _mm_body,
        grid=(E_PAD // MB,),
        in_specs=[
            pl.BlockSpec((MB, AUX), lambda i: (i, 0)),
            pl.BlockSpec((AUX, OUT), lambda i: (0, 0)),
            pl.BlockSpec((1, OUT), lambda i: (0, 0)),
        ],
        out_specs=[
            pl.BlockSpec((MB, HALF), lambda i: (i, 0)),
            pl.BlockSpec((MB, HALF), lambda i: (i, 0)),
        ],
        out_shape=[
            jax.ShapeDtypeStruct((E_PAD, HALF), _bf16),
            jax.ShapeDtypeStruct((E_PAD, HALF), _bf16),
        ],
    )(a, wt, b)


@functools.partial(
    pl.kernel,
    out_type=(
        jax.ShapeDtypeStruct((N_PAD, HALF), _f32),
        jax.ShapeDtypeStruct((N_PAD, HALF), _f32),
    ),
    mesh=_MESH,
    compiler_params=_NOTC,
    scratch_types=[
        pltpu.VMEM_SHARED((N_PAD, HALF), _f32),
        pltpu.VMEM((CC, HALF), _bf16),
        pltpu.VMEM((CC, HALF), _bf16),
        pltpu.VMEM((CC, HALF), _bf16),
        pltpu.VMEM((CC, HALF), _bf16),
        pltpu.VMEM((CC, HALF), _f32),
        pltpu.VMEM((CC, HALF), _f32),
        pltpu.VMEM((NCC, CC), jnp.int32),
        pltpu.VMEM((CC,), jnp.int32),
        pltpu.VMEM((CC,), jnp.int32),
        pltpu.VMEM((CC,), jnp.int32),
        pltpu.VMEM((CC,), jnp.int32),
        pltpu.VMEM((CC,), jnp.int32),
        pltpu.VMEM((CC,), jnp.int32),
        pltpu.SemaphoreType.DMA,
        pltpu.SemaphoreType.DMA,
        pltpu.SemaphoreType.DMA,
        pltpu.SemaphoreType.DMA,
        pltpu.SemaphoreType.DMA,
        pltpu.SemaphoreType.DMA,
    ],
)
def _stage_c(srclo_hbm, srchi_hbm, alo_hbm, ahi_hbm, packed_hbm,
             zero_hbm, outlo_hbm, outhi_hbm,
             acc_sh, r0, r1, a0, a1, p0, p1, packed_all,
             s0, s1, t0, t1, u0, u1,
             semr0, semr1, sema0, sema1, sems0, sems1):
    c = lax.axis_index("c")
    s = lax.axis_index("s")
    row0 = s * ROWS_PT
    pltpu.sync_copy(zero_hbm.at[pl.ds(row0, ROWS_PT)],
                    acc_sh.at[pl.ds(row0, ROWS_PT)])
    pltpu.sync_copy(packed_hbm.at[pl.ds(s * NCC, NCC)], packed_all)
    plsc.subcore_barrier()

    rbufs = (r0, r1)
    abufs = (a0, a1)
    pbufs = (p0, p1)
    sbufs = (s0, s1)
    tbufs = (t0, t1)
    ubufs = (u0, u1)
    rsems = (semr0, semr1)
    asems = (sema0, sema1)
    ssems = (sems0, sems1)

    def unpack_idx(i, b):
        for q in range(CC // L):
            sl = pl.ds(q * L, L)
            pk = packed_all[i, sl]
            sbufs[b][sl] = lax.shift_right_logical(pk, 14)
            tbufs[b][sl] = lax.bitwise_and(pk, PACK - 1)

    def run(tbl_hbm, a_half_hbm):
        def g_src(i, b):
            return tbl_hbm.at[sbufs[b]]

        def a_src(i):
            return a_half_hbm.at[pl.ds((s * NCC + i) * CC, CC)]

        unpack_idx(0, 0)
        pltpu.async_copy(g_src(0, 0), r0, semr0)
        pltpu.async_copy(a_src(0), a0, sema0)
        unpack_idx(1, 1)
        pltpu.async_copy(g_src(1, 1), r1, semr1)
        pltpu.async_copy(a_src(1), a1, sema1)

        def chunk(k, _):
            for b in (0, 1):
                i = 2 * k + b
                rb = rbufs[b]
                ab = abufs[b]
                pb = pbufs[b]
                ub = ubufs[b]
                pltpu.make_async_copy(g_src(i, b), rb, rsems[b]).wait()
                pltpu.make_async_copy(a_src(i), ab, asems[b]).wait()

                @pl.when(i >= 2)
                def _():
                    pltpu.make_async_copy(pb, acc_sh.at[ub],
                                          ssems[b]).wait()

                for q in range(CC // L):
                    sl = pl.ds(q * L, L)
                    ub[sl] = tbufs[b][sl]

                def row(j):
                    for q in range(HALF // 32):
                        sl32 = pl.ds(q * 32, 32)
                        rlo, rhi = plsc.unpack(
                            rb[j, sl32], format=plsc.PackFormat.INTERLEAVED)
                        alo, ahi = plsc.unpack(
                            ab[j, sl32], format=plsc.PackFormat.INTERLEAVED)
                        pb[j, pl.ds(q * 32, L)] = rlo * alo
                        pb[j, pl.ds(q * 32 + L, L)] = rhi * ahi

                _vloop(CC, row, unroll=2)
                pltpu.async_copy(pb, acc_sh.at[ub], ssems[b], add=True)

                @pl.when(i + 2 < NCC)
                def _():
                    unpack_idx(i + 2, b)
                    pltpu.async_copy(g_src(i + 2, b), rb, rsems[b])
                    pltpu.async_copy(a_src(i + 2), ab, asems[b])
            return 0

        lax.fori_loop(0, NCC // 2, chunk, 0, unroll=False)
        for b in (0, 1):
            pltpu.make_async_copy(pbufs[b], acc_sh.at[ubufs[b]],
                                  ssems[b]).wait()

    @pl.when(c == 0)
    def _():
        run(srclo_hbm, alo_hbm)

    @pl.when(c == 1)
    def _():
        run(srchi_hbm, ahi_hbm)

    plsc.subcore_barrier()

    @pl.when(c == 0)
    def _():
        pltpu.sync_copy(acc_sh.at[pl.ds(row0, ROWS_PT)],
                        outlo_hbm.at[pl.ds(row0, ROWS_PT)])

    @pl.when(c == 1)
    def _():
        pltpu.sync_copy(acc_sh.at[pl.ds(row0, ROWS_PT)],
                        outhi_hbm.at[pl.ds(row0, ROWS_PT)])


def kernel(src_ft, dst_ft, edge_p, edge_index, W_pos_w, W_pos_b):
    src = edge_index[0]
    dst = edge_index[1]
    pad = E_PAD - E
    p_pad = jnp.pad(edge_p, ((0, pad), (0, 0)))
    src_pad = jnp.pad(src, (0, pad))
    dst_pad = jnp.pad(dst, (0, pad), constant_values=N)
    dst2d = dst_pad.reshape(E_PAD // CH, CH)
    packed2d = (src_pad * PACK + dst_pad).reshape(E_PAD // CC, CC)
    zero_aux = jnp.zeros((N_PAD, AUX), _f32)
    zero_half = jnp.zeros((N_PAD, HALF), _f32)

    src_bf = src_ft.astype(_bf16)[:, _P256]
    wt_perm = W_pos_w.T[:, _P256]
    b_perm = W_pos_b[_P256]

    a = _stage_a(p_pad, dst2d, zero_aux)
    a_lo, a_hi = _linear(a, wt_perm, b_perm[None, :])
    out_lo, out_hi = _stage_c(src_bf[:, :HALF], src_bf[:, HALF:],
                              a_lo, a_hi, packed2d, zero_half)
    out = jnp.concatenate([out_lo[:N], out_hi[:N]], axis=1)
    return out[:, None, :]


# speedup vs baseline: 1.0001x; 1.0001x over previous
import functools

import jax
import jax.numpy as jnp
from jax import lax
from jax.experimental import pallas as pl
from jax.experimental.pallas import tpu as pltpu
from jax.experimental.pallas import tpu_sc as plsc

N = 10000
E = 160000
AUX = 16
OUT = 256
HALF = OUT // 2
NEG = 0.2

NC, NS, L = 2, 16, 16
NW = NC * NS
CH = 128
CC = 64
BLK = 512
SUB = BLK // CH
E_PAD = 163840
EPW = E_PAD // NW
EPT = E_PAD // NS
NB1 = EPT // BLK
NB2 = EPW // BLK
NCC = EPT // CC
PACK = 16384
N_PAD = 10112
ROWS_PT = N_PAD // NS

_MESH = plsc.VectorSubcoreMesh(core_axis_name="c", subcore_axis_name="s")
_f32 = jnp.float32
_bf16 = jnp.bfloat16
_NOTC = pltpu.CompilerParams(use_tc_tiling_on_sc=False,
                             needs_layout_passes=False)


def _leaky_exp(x):
    return jnp.exp(jnp.where(x >= 0, x, NEG * x))


def _vloop(n, body, unroll=8):
    def f(j, carry):
        body(j)
        return carry
    lax.fori_loop(0, n, f, 0, unroll=unroll)


@functools.partial(
    pl.kernel,
    out_type=jax.ShapeDtypeStruct((E_PAD, AUX), _f32),
    mesh=_MESH,
    compiler_params=_NOTC,
    scratch_types=[
        pltpu.VMEM_SHARED((N_PAD, AUX), _f32),
        pltpu.VMEM((BLK, AUX), _f32),
        pltpu.VMEM((BLK, AUX), _f32),
        pltpu.VMEM((BLK, AUX), _f32),
        pltpu.VMEM((BLK, AUX), _f32),
        pltpu.VMEM((NB1 * SUB, CH), jnp.int32),
        pltpu.VMEM((NB2 * SUB, CH), jnp.int32),
        pltpu.SemaphoreType.DMA,
        pltpu.SemaphoreType.DMA,
        pltpu.SemaphoreType.DMA,
        pltpu.SemaphoreType.DMA,
    ],
)
def _stage_a(p_hbm, dst2d_hbm, zero_hbm, a_hbm,
             den_sh, p0, p1, d0, d1, dstA, dstB,
             semp0, semp1, semg0, semg1):
    c = lax.axis_index("c")
    s = lax.axis_index("s")
    wid = s * NC + c
    row0 = s * ROWS_PT
    pltpu.sync_copy(zero_hbm.at[pl.ds(row0, ROWS_PT)],
                    den_sh.at[pl.ds(row0, ROWS_PT)])
    pltpu.sync_copy(dst2d_hbm.at[pl.ds(s * NB1 * SUB, NB1 * SUB)], dstA)
    pltpu.sync_copy(dst2d_hbm.at[pl.ds(wid * NB2 * SUB, NB2 * SUB)], dstB)
    plsc.subcore_barrier()

    pbufs = (p0, p1)
    dbufs = (d0, d1)
    psems = (semp0, semp1)
    gsems = (semg0, semg1)

    def p_src1(i):
        return p_hbm.at[pl.ds((s * NB1 + i) * BLK, BLK)]

    pltpu.async_copy(p_src1(0), p0, semp0)
    pltpu.async_copy(p_src1(1), p1, semp1)

    def blk1(k, _):
        for b in (0, 1):
            i = 2 * k + b
            pb = pbufs[b]
            pltpu.make_async_copy(p_src1(i), pb, psems[b]).wait()

            def row(j):
                pb[j, :] = _leaky_exp(pb[j, :])

            _vloop(BLK, row)
            for u in range(SUB):
                pltpu.sync_copy(pb.at[pl.ds(u * CH, CH)],
                                den_sh.at[dstA.at[i * SUB + u]], add=True)

            @pl.when(i + 2 < NB1)
            def _():
                pltpu.async_copy(p_src1(i + 2), pb, psems[b])
        return 0

    lax.fori_loop(0, NB1 // 2, blk1, 0, unroll=False)
    plsc.subcore_barrier()

    def p_src2(i):
        return p_hbm.at[pl.ds((wid * NB2 + i) * BLK, BLK)]

    def gathers(i, b):
        for u in range(SUB):
            pltpu.async_copy(den_sh.at[dstB.at[i * SUB + u]],
                             dbufs[b].at[pl.ds(u * CH, CH)], gsems[b])

    pltpu.async_copy(p_src2(0), p0, semp0)
    gathers(0, 0)
    pltpu.async_copy(p_src2(1), p1, semp1)
    gathers(1, 1)

    def blk2(k, _):
        for b in (0, 1):
            i = 2 * k + b
            pb = pbufs[b]
            db = dbufs[b]
            pltpu.make_async_copy(p_src2(i), pb, psems[b]).wait()
            for u in range(SUB):
                pltpu.make_async_copy(den_sh.at[dstB.at[i * SUB + u]],
                                      db.at[pl.ds(u * CH, CH)],
                                      gsems[b]).wait()

            def row(j):
                pb[j, :] = _leaky_exp(pb[j, :]) / db[j, :]

            _vloop(BLK, row)
            pltpu.sync_copy(pb, a_hbm.at[pl.ds((wid * NB2 + i) * BLK, BLK)])

            @pl.when(i + 2 < NB2)
            def _():
                pltpu.async_copy(p_src2(i + 2), pb, psems[b])
                gathers(i + 2, b)
        return 0

    lax.fori_loop(0, NB2 // 2, blk2, 0, unroll=False)


def _mm_body(a_ref, wt_ref, b_ref, lo_ref, hi_ref):
    r = jnp.dot(a_ref[...], wt_ref[...], preferred_element_type=_f32)
    r = (r + b_ref[...]).astype(_bf16)
    lo_ref[...] = r[:, :HALF]
    hi_ref[...] = r[:, HALF:]


def _linear(a, wt, b):
    MB = 1024
    return pl.pallas_call(
        _mm_body,
        grid=(E_PAD // MB,),
        in_specs=[
            pl.BlockSpec((MB, AUX), lambda i: (i, 0)),
            pl.BlockSpec((AUX, OUT), lambda i: (0, 0)),
            pl.BlockSpec((1, OUT), lambda i: (0, 0)),
        ],
        out_specs=[
            pl.BlockSpec((MB, HALF), lambda i: (i, 0)),
            pl.BlockSpec((MB, HALF), lambda i: (i, 0)),
        ],
        out_shape=[
            jax.ShapeDtypeStruct((E_PAD, HALF), _bf16),
            jax.ShapeDtypeStruct((E_PAD, HALF), _bf16),
        ],
    )(a, wt, b)


@functools.partial(
    pl.kernel,
    out_type=(
        jax.ShapeDtypeStruct((N_PAD, HALF), _f32),
        jax.ShapeDtypeStruct((N_PAD, HALF), _f32),
    ),
    mesh=_MESH,
    compiler_params=_NOTC,
    scratch_types=[
        pltpu.VMEM_SHARED((N_PAD, HALF), _f32),
        pltpu.VMEM((CC, HALF), _bf16),
        pltpu.VMEM((CC, HALF), _bf16),
        pltpu.VMEM((CC, HALF), _bf16),
        pltpu.VMEM((CC, HALF), _bf16),
        pltpu.VMEM((CC, HALF), _f32),
        pltpu.VMEM((CC, HALF), _f32),
        pltpu.VMEM((NCC, CC), jnp.int32),
        pltpu.VMEM((CC,), jnp.int32),
        pltpu.VMEM((CC,), jnp.int32),
        pltpu.VMEM((CC,), jnp.int32),
        pltpu.VMEM((CC,), jnp.int32),
        pltpu.VMEM((CC,), jnp.int32),
        pltpu.VMEM((CC,), jnp.int32),
        pltpu.SemaphoreType.DMA,
        pltpu.SemaphoreType.DMA,
        pltpu.SemaphoreType.DMA,
        pltpu.SemaphoreType.DMA,
        pltpu.SemaphoreType.DMA,
        pltpu.SemaphoreType.DMA,
    ],
)
def _stage_c(srclo_hbm, srchi_hbm, alo_hbm, ahi_hbm, packed_hbm,
             zero_hbm, outlo_hbm, outhi_hbm,
             acc_sh, r0, r1, a0, a1, p0, p1, packed_all,
             s0, s1, t0, t1, u0, u1,
             semr0, semr1, sema0, sema1, sems0, sems1):
    c = lax.axis_index("c")
    s = lax.axis_index("s")
    row0 = s * ROWS_PT
    pltpu.sync_copy(zero_hbm.at[pl.ds(row0, ROWS_PT)],
                    acc_sh.at[pl.ds(row0, ROWS_PT)])
    pltpu.sync_copy(packed_hbm.at[pl.ds(s * NCC, NCC)], packed_all)
    plsc.subcore_barrier()

    rbufs = (r0, r1)
    abufs = (a0, a1)
    pbufs = (p0, p1)
    sbufs = (s0, s1)
    tbufs = (t0, t1)
    ubufs = (u0, u1)
    rsems = (semr0, semr1)
    asems = (sema0, sema1)
    ssems = (sems0, sems1)

    def unpack_idx(i, b):
        for q in range(CC // L):
            sl = pl.ds(q * L, L)
            pk = packed_all[i, sl]
            sbufs[b][sl] = lax.shift_right_logical(pk, 14)
            tbufs[b][sl] = lax.bitwise_and(pk, PACK - 1)

    def run(tbl_hbm, a_half_hbm):
        def g_src(i, b):
            return tbl_hbm.at[sbufs[b]]

        def a_src(i):
            return a_half_hbm.at[pl.ds((s * NCC + i) * CC, CC)]

        unpack_idx(0, 0)
        pltpu.async_copy(g_src(0, 0), r0, semr0)
        pltpu.async_copy(a_src(0), a0, sema0)
        unpack_idx(1, 1)
        pltpu.async_copy(g_src(1, 1), r1, semr1)
        pltpu.async_copy(a_src(1), a1, sema1)

        ev = lax.iota(jnp.int32, L) * 2

        def chunk(k, _):
            for b in (0, 1):
                i = 2 * k + b
                rb = rbufs[b]
                ab = abufs[b]
                pb = pbufs[b]
                ub = ubufs[b]
                pltpu.make_async_copy(g_src(i, b), rb, rsems[b]).wait()
                pltpu.make_async_copy(a_src(i), ab, asems[b]).wait()

                @pl.when(i >= 2)
                def _():
                    pltpu.make_async_copy(pb, acc_sh.at[ub],
                                          ssems[b]).wait()

                for q in range(CC // L):
                    sl = pl.ds(q * L, L)
                    ub[sl] = tbufs[b][sl]

                def row(j):
                    jv = jnp.full((L,), j, dtype=jnp.int32)
                    for q in range(HALF // 32):
                        sl32 = pl.ds(q * 32, 32)
                        rlo, rhi = plsc.unpack(
                            rb[j, sl32], format=plsc.PackFormat.INTERLEAVED)
                        alo, ahi = plsc.unpack(
                            ab[j, sl32], format=plsc.PackFormat.INTERLEAVED)
                        plsc.store_scatter(pb, [jv, ev + q * 32], rlo * alo)
                        plsc.store_scatter(pb, [jv, ev + q * 32 + 1],
                                           rhi * ahi)

                _vloop(CC, row, unroll=2)
                pltpu.async_copy(pb, acc_sh.at[ub], ssems[b], add=True)

                @pl.when(i + 2 < NCC)
                def _():
                    unpack_idx(i + 2, b)
                    pltpu.async_copy(g_src(i + 2, b), rb, rsems[b])
                    pltpu.async_copy(a_src(i + 2), ab, asems[b])
            return 0

        lax.fori_loop(0, NCC // 2, chunk, 0, unroll=False)
        for b in (0, 1):
            pltpu.make_async_copy(pbufs[b], acc_sh.at[ubufs[b]],
                                  ssems[b]).wait()

    @pl.when(c == 0)
    def _():
        run(srclo_hbm, alo_hbm)

    @pl.when(c == 1)
    def _():
        run(srchi_hbm, ahi_hbm)

    plsc.subcore_barrier()

    @pl.when(c == 0)
    def _():
        pltpu.sync_copy(acc_sh.at[pl.ds(row0, ROWS_PT)],
                        outlo_hbm.at[pl.ds(row0, ROWS_PT)])

    @pl.when(c == 1)
    def _():
        pltpu.sync_copy(acc_sh.at[pl.ds(row0, ROWS_PT)],
                        outhi_hbm.at[pl.ds(row0, ROWS_PT)])


def kernel(src_ft, dst_ft, edge_p, edge_index, W_pos_w, W_pos_b):
    src = edge_index[0]
    dst = edge_index[1]
    pad = E_PAD - E
    p_pad = jnp.pad(edge_p, ((0, pad), (0, 0)))
    src_pad = jnp.pad(src, (0, pad))
    dst_pad = jnp.pad(dst, (0, pad), constant_values=N)
    dst2d = dst_pad.reshape(E_PAD // CH, CH)
    packed2d = (src_pad * PACK + dst_pad).reshape(E_PAD // CC, CC)
    zero_aux = jnp.zeros((N_PAD, AUX), _f32)
    zero_half = jnp.zeros((N_PAD, HALF), _f32)

    src_bf = src_ft.astype(_bf16)

    a = _stage_a(p_pad, dst2d, zero_aux)
    a_lo, a_hi = _linear(a, W_pos_w.T, W_pos_b[None, :])
    out_lo, out_hi = _stage_c(src_bf[:, :HALF], src_bf[:, HALF:],
                              a_lo, a_hi, packed2d, zero_half)
    out = jnp.concatenate([out_lo[:N], out_hi[:N]], axis=1)
    return out[:, None, :]


# speedup vs baseline: 1.0825x; 1.0824x over previous
import functools

import jax
import jax.numpy as jnp
from jax import lax
from jax.experimental import pallas as pl
from jax.experimental.pallas import tpu as pltpu
from jax.experimental.pallas import tpu_sc as plsc

N = 10000
E = 160000
AUX = 16
OUT = 256
HALF = OUT // 2
NEG = 0.2

NC, NS, L = 2, 16, 16
NW = NC * NS
CH = 128
CC = 64
BLK = 512
SUB = BLK // CH
E_PAD = 163840
EPW = E_PAD // NW
EPT = E_PAD // NS
NB1 = EPT // BLK
NB2 = EPW // BLK
NCC = EPT // CC
PACK = 16384
N_PAD = 10112
ROWS_PT = N_PAD // NS

_MESH = plsc.VectorSubcoreMesh(core_axis_name="c", subcore_axis_name="s")
_f32 = jnp.float32
_bf16 = jnp.bfloat16
_NOTC = pltpu.CompilerParams(use_tc_tiling_on_sc=False,
                             needs_layout_passes=False)
_TILED = pltpu.CompilerParams(use_tc_tiling_on_sc=True,
                              needs_layout_passes=False)


def _leaky_exp(x):
    return jnp.exp(jnp.where(x >= 0, x, NEG * x))


def _vloop(n, body, unroll=8):
    def f(j, carry):
        body(j)
        return carry
    lax.fori_loop(0, n, f, 0, unroll=unroll)


@functools.partial(
    pl.kernel,
    out_type=jax.ShapeDtypeStruct((E_PAD, AUX), _f32),
    mesh=_MESH,
    compiler_params=_NOTC,
    scratch_types=[
        pltpu.VMEM_SHARED((N_PAD, AUX), _f32),
        pltpu.VMEM((BLK, AUX), _f32),
        pltpu.VMEM((BLK, AUX), _f32),
        pltpu.VMEM((BLK, AUX), _f32),
        pltpu.VMEM((BLK, AUX), _f32),
        pltpu.VMEM((NB1 * SUB, CH), jnp.int32),
        pltpu.VMEM((NB2 * SUB, CH), jnp.int32),
        pltpu.SemaphoreType.DMA,
        pltpu.SemaphoreType.DMA,
        pltpu.SemaphoreType.DMA,
        pltpu.SemaphoreType.DMA,
    ],
)
def _stage_a(p_hbm, dst2d_hbm, zero_hbm, a_hbm,
             den_sh, p0, p1, d0, d1, dstA, dstB,
             semp0, semp1, semg0, semg1):
    c = lax.axis_index("c")
    s = lax.axis_index("s")
    wid = s * NC + c
    row0 = s * ROWS_PT
    pltpu.sync_copy(zero_hbm.at[pl.ds(row0, ROWS_PT)],
                    den_sh.at[pl.ds(row0, ROWS_PT)])
    pltpu.sync_copy(dst2d_hbm.at[pl.ds(s * NB1 * SUB, NB1 * SUB)], dstA)
    pltpu.sync_copy(dst2d_hbm.at[pl.ds(wid * NB2 * SUB, NB2 * SUB)], dstB)
    plsc.subcore_barrier()

    pbufs = (p0, p1)
    dbufs = (d0, d1)
    psems = (semp0, semp1)
    gsems = (semg0, semg1)

    def p_src1(i):
        return p_hbm.at[pl.ds((s * NB1 + i) * BLK, BLK)]

    pltpu.async_copy(p_src1(0), p0, semp0)
    pltpu.async_copy(p_src1(1), p1, semp1)

    def blk1(k, _):
        for b in (0, 1):
            i = 2 * k + b
            pb = pbufs[b]
            pltpu.make_async_copy(p_src1(i), pb, psems[b]).wait()

            def row(j):
                pb[j, :] = _leaky_exp(pb[j, :])

            _vloop(BLK, row)
            for u in range(SUB):
                pltpu.sync_copy(pb.at[pl.ds(u * CH, CH)],
                                den_sh.at[dstA.at[i * SUB + u]], add=True)

            @pl.when(i + 2 < NB1)
            def _():
                pltpu.async_copy(p_src1(i + 2), pb, psems[b])
        return 0

    lax.fori_loop(0, NB1 // 2, blk1, 0, unroll=False)
    plsc.subcore_barrier()

    def p_src2(i):
        return p_hbm.at[pl.ds((wid * NB2 + i) * BLK, BLK)]

    def gathers(i, b):
        for u in range(SUB):
            pltpu.async_copy(den_sh.at[dstB.at[i * SUB + u]],
                             dbufs[b].at[pl.ds(u * CH, CH)], gsems[b])

    pltpu.async_copy(p_src2(0), p0, semp0)
    gathers(0, 0)
    pltpu.async_copy(p_src2(1), p1, semp1)
    gathers(1, 1)

    def blk2(k, _):
        for b in (0, 1):
            i = 2 * k + b
            pb = pbufs[b]
            db = dbufs[b]
            pltpu.make_async_copy(p_src2(i), pb, psems[b]).wait()
            for u in range(SUB):
                pltpu.make_async_copy(den_sh.at[dstB.at[i * SUB + u]],
                                      db.at[pl.ds(u * CH, CH)],
                                      gsems[b]).wait()

            def row(j):
                pb[j, :] = _leaky_exp(pb[j, :]) / db[j, :]

            _vloop(BLK, row)
            pltpu.sync_copy(pb, a_hbm.at[pl.ds((wid * NB2 + i) * BLK, BLK)])

            @pl.when(i + 2 < NB2)
            def _():
                pltpu.async_copy(p_src2(i + 2), pb, psems[b])
                gathers(i + 2, b)
        return 0

    lax.fori_loop(0, NB2 // 2, blk2, 0, unroll=False)


def _mm_body(a_ref, wt_ref, b_ref, lo_ref, hi_ref):
    r = jnp.dot(a_ref[...], wt_ref[...], preferred_element_type=_f32)
    r = r + b_ref[...]
    lo_ref[...] = r[:, :HALF]
    hi_ref[...] = r[:, HALF:]


def _linear(a, wt, b):
    MB = 1024
    return pl.pallas_call(
        _mm_body,
        grid=(E_PAD // MB,),
        in_specs=[
            pl.BlockSpec((MB, AUX), lambda i: (i, 0)),
            pl.BlockSpec((AUX, OUT), lambda i: (0, 0)),
            pl.BlockSpec((1, OUT), lambda i: (0, 0)),
        ],
        out_specs=[
            pl.BlockSpec((MB, HALF), lambda i: (i, 0)),
            pl.BlockSpec((MB, HALF), lambda i: (i, 0)),
        ],
        out_shape=[
            jax.ShapeDtypeStruct((E_PAD, HALF), _f32),
            jax.ShapeDtypeStruct((E_PAD, HALF), _f32),
        ],
    )(a, wt, b)


@functools.partial(
    pl.kernel,
    out_type=(
        jax.ShapeDtypeStruct((N_PAD, HALF), _f32),
        jax.ShapeDtypeStruct((N_PAD, HALF), _f32),
    ),
    mesh=_MESH,
    compiler_params=_TILED,
    scratch_types=[
        pltpu.VMEM_SHARED((N_PAD, HALF), _f32),
        pltpu.VMEM((CC, HALF), _f32),
        pltpu.VMEM((CC, HALF), _f32),
        pltpu.VMEM((CC, HALF), _f32),
        pltpu.VMEM((CC, HALF), _f32),
        pltpu.VMEM((CC, HALF), _f32),
        pltpu.VMEM((CC, HALF), _f32),
        pltpu.VMEM((CC,), jnp.int32),
        pltpu.VMEM((CC,), jnp.int32),
        pltpu.VMEM((CC,), jnp.int32),
        pltpu.VMEM((CC,), jnp.int32),
        pltpu.VMEM((CC,), jnp.int32),
        pltpu.VMEM((CC,), jnp.int32),
        pltpu.VMEM((CC,), jnp.int32),
        pltpu.VMEM((CC,), jnp.int32),
        pltpu.SemaphoreType.DMA,
        pltpu.SemaphoreType.DMA,
        pltpu.SemaphoreType.DMA,
        pltpu.SemaphoreType.DMA,
        pltpu.SemaphoreType.DMA,
        pltpu.SemaphoreType.DMA,
        pltpu.SemaphoreType.DMA,
        pltpu.SemaphoreType.DMA,
    ],
)
def _stage_c(srclo_hbm, srchi_hbm, alo_hbm, ahi_hbm, packed_hbm,
             zero_hbm, outlo_hbm, outhi_hbm,
             acc_sh, r0, r1, a0, a1, p0, p1, k0, k1,
             s0, s1, t0, t1, u0, u1,
             semr0, semr1, sema0, sema1, sems0, sems1, semk0, semk1):
    c = lax.axis_index("c")
    s = lax.axis_index("s")
    row0 = s * ROWS_PT
    pltpu.sync_copy(zero_hbm.at[pl.ds(row0, ROWS_PT)],
                    acc_sh.at[pl.ds(row0, ROWS_PT)])
    plsc.subcore_barrier()

    rbufs = (r0, r1)
    abufs = (a0, a1)
    pbufs = (p0, p1)
    kbufs = (k0, k1)
    sbufs = (s0, s1)
    tbufs = (t0, t1)
    ubufs = (u0, u1)
    rsems = (semr0, semr1)
    asems = (sema0, sema1)
    ssems = (sems0, sems1)
    ksems = (semk0, semk1)

    def k_src(i):
        return packed_hbm.at[pl.ds((s * NCC + i) * CC, CC)]

    def unpack_idx(i, b):
        for q in range(CC // L):
            sl = pl.ds(q * L, L)
            pk = kbufs[b][sl]
            sbufs[b][sl] = lax.shift_right_logical(pk, 14)
            tbufs[b][sl] = lax.bitwise_and(pk, PACK - 1)

    def run(tbl_hbm, a_half_hbm):
        def g_src(i, b):
            return tbl_hbm.at[sbufs[b]]

        def a_src(i):
            return a_half_hbm.at[pl.ds((s * NCC + i) * CC, CC)]

        pltpu.async_copy(k_src(0), k0, semk0)
        pltpu.async_copy(k_src(1), k1, semk1)
        pltpu.make_async_copy(k_src(0), k0, semk0).wait()
        unpack_idx(0, 0)
        pltpu.async_copy(g_src(0, 0), r0, semr0)
        pltpu.async_copy(a_src(0), a0, sema0)
        pltpu.async_copy(k_src(2), k0, semk0)
        pltpu.make_async_copy(k_src(1), k1, semk1).wait()
        unpack_idx(1, 1)
        pltpu.async_copy(g_src(1, 1), r1, semr1)
        pltpu.async_copy(a_src(1), a1, sema1)
        pltpu.async_copy(k_src(3), k1, semk1)

        def chunk(k, _):
            for b in (0, 1):
                i = 2 * k + b
                rb = rbufs[b]
                ab = abufs[b]
                pb = pbufs[b]
                ub = ubufs[b]
                pltpu.make_async_copy(g_src(i, b), rb, rsems[b]).wait()
                pltpu.make_async_copy(a_src(i), ab, asems[b]).wait()

                @pl.when(i >= 2)
                def _():
                    pltpu.make_async_copy(pb, acc_sh.at[ub],
                                          ssems[b]).wait()

                for q in range(CC // L):
                    sl = pl.ds(q * L, L)
                    ub[sl] = tbufs[b][sl]

                def row(j):
                    for q in range(HALF // L):
                        sl = pl.ds(q * L, L)
                        pb[j, sl] = rb[j, sl] * ab[j, sl]

                _vloop(CC, row, unroll=2)
                pltpu.async_copy(pb, acc_sh.at[ub], ssems[b], add=True)

                @pl.when(i + 2 < NCC)
                def _():
                    pltpu.make_async_copy(k_src(i + 2), kbufs[b],
                                          ksems[b]).wait()
                    unpack_idx(i + 2, b)
                    pltpu.async_copy(g_src(i + 2, b), rb, rsems[b])
                    pltpu.async_copy(a_src(i + 2), ab, asems[b])

                    @pl.when(i + 4 < NCC)
                    def _():
                        pltpu.async_copy(k_src(i + 4), kbufs[b], ksems[b])
            return 0

        lax.fori_loop(0, NCC // 2, chunk, 0, unroll=False)
        for b in (0, 1):
            pltpu.make_async_copy(pbufs[b], acc_sh.at[ubufs[b]],
                                  ssems[b]).wait()

    @pl.when(c == 0)
    def _():
        run(srclo_hbm, alo_hbm)

    @pl.when(c == 1)
    def _():
        run(srchi_hbm, ahi_hbm)

    plsc.subcore_barrier()

    @pl.when(c == 0)
    def _():
        pltpu.sync_copy(acc_sh.at[pl.ds(row0, ROWS_PT)],
                        outlo_hbm.at[pl.ds(row0, ROWS_PT)])

    @pl.when(c == 1)
    def _():
        pltpu.sync_copy(acc_sh.at[pl.ds(row0, ROWS_PT)],
                        outhi_hbm.at[pl.ds(row0, ROWS_PT)])


def kernel(src_ft, dst_ft, edge_p, edge_index, W_pos_w, W_pos_b):
    src = edge_index[0]
    dst = edge_index[1]
    pad = E_PAD - E
    p_pad = jnp.pad(edge_p, ((0, pad), (0, 0)))
    src_pad = jnp.pad(src, (0, pad))
    dst_pad = jnp.pad(dst, (0, pad), constant_values=N)
    dst2d = dst_pad.reshape(E_PAD // CH, CH)
    packed1d = src_pad * PACK + dst_pad
    zero_aux = jnp.zeros((N_PAD, AUX), _f32)
    zero_half = jnp.zeros((N_PAD, HALF), _f32)

    a = _stage_a(p_pad, dst2d, zero_aux)
    a_lo, a_hi = _linear(a, W_pos_w.T, W_pos_b[None, :])
    out_lo, out_hi = _stage_c(src_ft[:, :HALF], src_ft[:, HALF:],
                              a_lo, a_hi, packed1d, zero_half)
    out = jnp.concatenate([out_lo[:N], out_hi[:N]], axis=1)
    return out[:, None, :]


# speedup vs baseline: 1.0993x; 1.0155x over previous
import functools

import jax
import jax.numpy as jnp
from jax import lax
from jax.experimental import pallas as pl
from jax.experimental.pallas import tpu as pltpu
from jax.experimental.pallas import tpu_sc as plsc

N = 10000
E = 160000
AUX = 16
OUT = 256
HALF = OUT // 2
NEG = 0.2

NC, NS, L = 2, 16, 16
NW = NC * NS
CH = 128
CC = 64
BLK = 512
SUB = BLK // CH
E_PAD = 163840
EPW = E_PAD // NW
EPT = E_PAD // NS
NB1 = EPT // BLK
NB2 = EPW // BLK
NCC = EPT // CC
PACK = 16384
N_PAD = 10112
ROWS_PT = N_PAD // NS

_MESH = plsc.VectorSubcoreMesh(core_axis_name="c", subcore_axis_name="s")
_f32 = jnp.float32
_bf16 = jnp.bfloat16
_NOTC = pltpu.CompilerParams(use_tc_tiling_on_sc=False,
                             needs_layout_passes=False)
_TILED = pltpu.CompilerParams(use_tc_tiling_on_sc=True,
                              needs_layout_passes=False)


def _leaky_exp(x):
    return jnp.exp(jnp.where(x >= 0, x, NEG * x))


def _vloop(n, body, unroll=8):
    def f(j, carry):
        body(j)
        return carry
    lax.fori_loop(0, n, f, 0, unroll=unroll)


@functools.partial(
    pl.kernel,
    out_type=jax.ShapeDtypeStruct((E_PAD // 8, 128), _f32),
    mesh=_MESH,
    compiler_params=_NOTC,
    scratch_types=[
        pltpu.VMEM_SHARED((N_PAD, AUX), _f32),
        pltpu.VMEM((BLK // 8, 128), _f32),
        pltpu.VMEM((BLK // 8, 128), _f32),
        pltpu.VMEM((BLK, AUX), _f32),
        pltpu.VMEM((BLK, AUX), _f32),
        pltpu.VMEM((BLK, AUX), _f32),
        pltpu.VMEM((BLK // 8, 128), _f32),
        pltpu.VMEM((NB1 * SUB, CH), jnp.int32),
        pltpu.VMEM((NB2 * SUB, CH), jnp.int32),
        pltpu.SemaphoreType.DMA,
        pltpu.SemaphoreType.DMA,
        pltpu.SemaphoreType.DMA,
        pltpu.SemaphoreType.DMA,
    ],
)
def _stage_a(p_hbm, dst2d_hbm, zero_hbm, a_hbm,
             den_sh, p0, p1, d0, d1, sv, av, dstA, dstB,
             semp0, semp1, semg0, semg1):
    c = lax.axis_index("c")
    s = lax.axis_index("s")
    wid = s * NC + c
    row0 = s * ROWS_PT
    pltpu.sync_copy(zero_hbm.at[pl.ds(row0, ROWS_PT)],
                    den_sh.at[pl.ds(row0, ROWS_PT)])
    pltpu.sync_copy(dst2d_hbm.at[pl.ds(s * NB1 * SUB, NB1 * SUB)], dstA)
    pltpu.sync_copy(dst2d_hbm.at[pl.ds(wid * NB2 * SUB, NB2 * SUB)], dstB)
    plsc.subcore_barrier()

    pbufs = (p0, p1)
    dbufs = (d0, d1)
    psems = (semp0, semp1)
    gsems = (semg0, semg1)

    def p_src1(i):
        return p_hbm.at[pl.ds((s * NB1 + i) * (BLK // 8), BLK // 8)]

    pltpu.async_copy(p_src1(0), p0, semp0)
    pltpu.async_copy(p_src1(1), p1, semp1)

    def blk1(k, _):
        for b in (0, 1):
            i = 2 * k + b
            pb = pbufs[b]
            pltpu.make_async_copy(p_src1(i), pb, psems[b]).wait()

            def row(r):
                for q in range(8):
                    sv[r * 8 + q, :] = _leaky_exp(pb[r, pl.ds(q * L, L)])

            _vloop(BLK // 8, row)
            for u in range(SUB):
                pltpu.sync_copy(sv.at[pl.ds(u * CH, CH)],
                                den_sh.at[dstA.at[i * SUB + u]], add=True)

            @pl.when(i + 2 < NB1)
            def _():
                pltpu.async_copy(p_src1(i + 2), pb, psems[b])
        return 0

    lax.fori_loop(0, NB1 // 2, blk1, 0, unroll=False)
    plsc.subcore_barrier()

    def p_src2(i):
        return p_hbm.at[pl.ds((wid * NB2 + i) * (BLK // 8), BLK // 8)]

    def gathers(i, b):
        for u in range(SUB):
            pltpu.async_copy(den_sh.at[dstB.at[i * SUB + u]],
                             dbufs[b].at[pl.ds(u * CH, CH)], gsems[b])

    pltpu.async_copy(p_src2(0), p0, semp0)
    gathers(0, 0)
    pltpu.async_copy(p_src2(1), p1, semp1)
    gathers(1, 1)

    def blk2(k, _):
        for b in (0, 1):
            i = 2 * k + b
            pb = pbufs[b]
            db = dbufs[b]
            pltpu.make_async_copy(p_src2(i), pb, psems[b]).wait()
            for u in range(SUB):
                pltpu.make_async_copy(den_sh.at[dstB.at[i * SUB + u]],
                                      db.at[pl.ds(u * CH, CH)],
                                      gsems[b]).wait()

            def row(r):
                for q in range(8):
                    sl = pl.ds(q * L, L)
                    av[r, sl] = _leaky_exp(pb[r, sl]) / db[r * 8 + q, :]

            _vloop(BLK // 8, row)
            pltpu.sync_copy(
                av, a_hbm.at[pl.ds((wid * NB2 + i) * (BLK // 8), BLK // 8)])

            @pl.when(i + 2 < NB2)
            def _():
                pltpu.async_copy(p_src2(i + 2), pb, psems[b])
                gathers(i + 2, b)
        return 0

    lax.fori_loop(0, NB2 // 2, blk2, 0, unroll=False)


def _mm_body(a_ref, wlo_ref, whi_ref, blo_ref, bhi_ref, lo_ref, hi_ref):
    a = a_ref[...]
    lo_ref[...] = jnp.dot(a, wlo_ref[...],
                          preferred_element_type=_f32) + blo_ref[...]
    hi_ref[...] = jnp.dot(a, whi_ref[...],
                          preferred_element_type=_f32) + bhi_ref[...]


def _linear(a2d, wt, b):
    wexp = jnp.zeros((8, AUX, 8, HALF), _f32)
    wlo = wexp.at[jnp.arange(8), :, jnp.arange(8), :].set(wt[:, :HALF])
    whi = wexp.at[jnp.arange(8), :, jnp.arange(8), :].set(wt[:, HALF:])
    wlo = wlo.reshape(128, 8 * HALF)
    whi = whi.reshape(128, 8 * HALF)
    blo = jnp.tile(b[:HALF], 8)[None, :]
    bhi = jnp.tile(b[HALF:], 8)[None, :]
    MB = 128
    outs = pl.pallas_call(
        _mm_body,
        grid=(E_PAD // 8 // MB,),
        in_specs=[
            pl.BlockSpec((MB, 128), lambda i: (i, 0)),
            pl.BlockSpec((128, 8 * HALF), lambda i: (0, 0)),
            pl.BlockSpec((128, 8 * HALF), lambda i: (0, 0)),
            pl.BlockSpec((1, 8 * HALF), lambda i: (0, 0)),
            pl.BlockSpec((1, 8 * HALF), lambda i: (0, 0)),
        ],
        out_specs=[
            pl.BlockSpec((MB, 8 * HALF), lambda i: (i, 0)),
            pl.BlockSpec((MB, 8 * HALF), lambda i: (i, 0)),
        ],
        out_shape=[
            jax.ShapeDtypeStruct((E_PAD // 8, 8 * HALF), _f32),
            jax.ShapeDtypeStruct((E_PAD // 8, 8 * HALF), _f32),
        ],
    )(a2d, wlo, whi, blo, bhi)
    return outs


@functools.partial(
    pl.kernel,
    out_type=(
        jax.ShapeDtypeStruct((N_PAD, HALF), _f32),
        jax.ShapeDtypeStruct((N_PAD, HALF), _f32),
    ),
    mesh=_MESH,
    compiler_params=_TILED,
    scratch_types=[
        pltpu.VMEM_SHARED((N_PAD, HALF), _f32),
        pltpu.VMEM((CC, HALF), _f32),
        pltpu.VMEM((CC, HALF), _f32),
        pltpu.VMEM((CC // 8, 8 * HALF), _f32),
        pltpu.VMEM((CC // 8, 8 * HALF), _f32),
        pltpu.VMEM((CC, HALF), _f32),
        pltpu.VMEM((CC, HALF), _f32),
        pltpu.VMEM((CC,), jnp.int32),
        pltpu.VMEM((CC,), jnp.int32),
        pltpu.VMEM((CC,), jnp.int32),
        pltpu.VMEM((CC,), jnp.int32),
        pltpu.VMEM((CC,), jnp.int32),
        pltpu.VMEM((CC,), jnp.int32),
        pltpu.VMEM((CC,), jnp.int32),
        pltpu.VMEM((CC,), jnp.int32),
        pltpu.SemaphoreType.DMA,
        pltpu.SemaphoreType.DMA,
        pltpu.SemaphoreType.DMA,
        pltpu.SemaphoreType.DMA,
        pltpu.SemaphoreType.DMA,
        pltpu.SemaphoreType.DMA,
        pltpu.SemaphoreType.DMA,
        pltpu.SemaphoreType.DMA,
    ],
)
def _stage_c(srclo_hbm, srchi_hbm, alo_hbm, ahi_hbm, packed_hbm,
             zero_hbm, outlo_hbm, outhi_hbm,
             acc_sh, r0, r1, a0, a1, p0, p1, k0, k1,
             s0, s1, t0, t1, u0, u1,
             semr0, semr1, sema0, sema1, sems0, sems1, semk0, semk1):
    c = lax.axis_index("c")
    s = lax.axis_index("s")
    row0 = s * ROWS_PT
    pltpu.sync_copy(zero_hbm.at[pl.ds(row0, ROWS_PT)],
                    acc_sh.at[pl.ds(row0, ROWS_PT)])
    plsc.subcore_barrier()

    rbufs = (r0, r1)
    abufs = (a0, a1)
    pbufs = (p0, p1)
    kbufs = (k0, k1)
    sbufs = (s0, s1)
    tbufs = (t0, t1)
    ubufs = (u0, u1)
    rsems = (semr0, semr1)
    asems = (sema0, sema1)
    ssems = (sems0, sems1)
    ksems = (semk0, semk1)

    def k_src(i):
        return packed_hbm.at[pl.ds((s * NCC + i) * CC, CC)]

    def unpack_idx(i, b):
        for q in range(CC // L):
            sl = pl.ds(q * L, L)
            pk = kbufs[b][sl]
            sbufs[b][sl] = lax.shift_right_logical(pk, 14)
            tbufs[b][sl] = lax.bitwise_and(pk, PACK - 1)

    def run(tbl_hbm, a_half_hbm):
        def g_src(i, b):
            return tbl_hbm.at[sbufs[b]]

        def a_src(i):
            return a_half_hbm.at[pl.ds((s * NCC + i) * (CC // 8), CC // 8)]

        pltpu.async_copy(k_src(0), k0, semk0)
        pltpu.async_copy(k_src(1), k1, semk1)
        pltpu.make_async_copy(k_src(0), k0, semk0).wait()
        unpack_idx(0, 0)
        pltpu.async_copy(g_src(0, 0), r0, semr0)
        pltpu.async_copy(a_src(0), a0, sema0)
        pltpu.async_copy(k_src(2), k0, semk0)
        pltpu.make_async_copy(k_src(1), k1, semk1).wait()
        unpack_idx(1, 1)
        pltpu.async_copy(g_src(1, 1), r1, semr1)
        pltpu.async_copy(a_src(1), a1, sema1)
        pltpu.async_copy(k_src(3), k1, semk1)

        def chunk(k, _):
            for b in (0, 1):
                i = 2 * k + b
                rb = rbufs[b]
                ab = abufs[b]
                pb = pbufs[b]
                ub = ubufs[b]
                pltpu.make_async_copy(g_src(i, b), rb, rsems[b]).wait()
                pltpu.make_async_copy(a_src(i), ab, asems[b]).wait()

                @pl.when(i >= 2)
                def _():
                    pltpu.make_async_copy(pb, acc_sh.at[ub],
                                          ssems[b]).wait()

                for q in range(CC // L):
                    sl = pl.ds(q * L, L)
                    ub[sl] = tbufs[b][sl]

                def row(m):
                    for e8 in range(8):
                        j = m * 8 + e8
                        for q in range(HALF // L):
                            pb[j, pl.ds(q * L, L)] = (
                                rb[j, pl.ds(q * L, L)]
                                * ab[m, pl.ds(e8 * HALF + q * L, L)])

                _vloop(CC // 8, row, unroll=1)
                pltpu.async_copy(pb, acc_sh.at[ub], ssems[b], add=True)

                @pl.when(i + 2 < NCC)
                def _():
                    pltpu.make_async_copy(k_src(i + 2), kbufs[b],
                                          ksems[b]).wait()
                    unpack_idx(i + 2, b)
                    pltpu.async_copy(g_src(i + 2, b), rb, rsems[b])
                    pltpu.async_copy(a_src(i + 2), ab, asems[b])

                    @pl.when(i + 4 < NCC)
                    def _():
                        pltpu.async_copy(k_src(i + 4), kbufs[b], ksems[b])
            return 0

        lax.fori_loop(0, NCC // 2, chunk, 0, unroll=False)
        for b in (0, 1):
            pltpu.make_async_copy(pbufs[b], acc_sh.at[ubufs[b]],
                                  ssems[b]).wait()

    @pl.when(c == 0)
    def _():
        run(srclo_hbm, alo_hbm)

    @pl.when(c == 1)
    def _():
        run(srchi_hbm, ahi_hbm)

    plsc.subcore_barrier()

    @pl.when(c == 0)
    def _():
        pltpu.sync_copy(acc_sh.at[pl.ds(row0, ROWS_PT)],
                        outlo_hbm.at[pl.ds(row0, ROWS_PT)])

    @pl.when(c == 1)
    def _():
        pltpu.sync_copy(acc_sh.at[pl.ds(row0, ROWS_PT)],
                        outhi_hbm.at[pl.ds(row0, ROWS_PT)])


def kernel(src_ft, dst_ft, edge_p, edge_index, W_pos_w, W_pos_b):
    src = edge_index[0]
    dst = edge_index[1]
    pad = E_PAD - E
    p2d = jnp.pad(edge_p.reshape(E // 8, 128), ((0, pad // 8), (0, 0)))
    src_pad = jnp.pad(src, (0, pad))
    dst_pad = jnp.pad(dst, (0, pad), constant_values=N)
    dst2d = dst_pad.reshape(E_PAD // CH, CH)
    packed1d = src_pad * PACK + dst_pad
    zero_aux = jnp.zeros((N_PAD, AUX), _f32)
    zero_half = jnp.zeros((N_PAD, HALF), _f32)

    a2d = _stage_a(p2d, dst2d, zero_aux)
    a_lo, a_hi = _linear(a2d, W_pos_w.T, W_pos_b)
    out_lo, out_hi = _stage_c(src_ft[:, :HALF], src_ft[:, HALF:],
                              a_lo, a_hi, packed1d, zero_half)
    out = jnp.concatenate([out_lo[:N], out_hi[:N]], axis=1)
    return out[:, None, :]


# speedup vs baseline: 1.3577x; 1.2351x over previous
import functools

import jax
import jax.numpy as jnp
from jax import lax
from jax.experimental import pallas as pl
from jax.experimental.pallas import tpu as pltpu
from jax.experimental.pallas import tpu_sc as plsc

N = 10000
E = 160000
AUX = 16
OUT = 256
HALF = OUT // 2
NEG = 0.2

NC, NS, L = 2, 16, 16
NW = NC * NS
CH = 128
CC = 64
BLK = 512
SUB = BLK // CH
E_PAD = 163840
EPW = E_PAD // NW
EPT = E_PAD // NS
NB1 = EPT // BLK
NB2 = EPW // BLK
NCC = EPT // CC
PACK = 16384
N_PAD = 10112
ROWS_PT = N_PAD // NS

_MESH = plsc.VectorSubcoreMesh(core_axis_name="c", subcore_axis_name="s")
_f32 = jnp.float32
_bf16 = jnp.bfloat16
_NOTC = pltpu.CompilerParams(use_tc_tiling_on_sc=False,
                             needs_layout_passes=False)
_TILED = pltpu.CompilerParams(use_tc_tiling_on_sc=True,
                              needs_layout_passes=False)


def _leaky_exp(x):
    return jnp.exp(jnp.where(x >= 0, x, NEG * x))


def _vloop(n, body, unroll=8):
    def f(j, carry):
        body(j)
        return carry
    lax.fori_loop(0, n, f, 0, unroll=unroll)


@functools.partial(
    pl.kernel,
    out_type=jax.ShapeDtypeStruct((E_PAD // 8, 128), _f32),
    mesh=_MESH,
    compiler_params=_NOTC,
    scratch_types=[
        pltpu.VMEM_SHARED((N_PAD, AUX), _f32),
        pltpu.VMEM((BLK // 8, 128), _f32),
        pltpu.VMEM((BLK // 8, 128), _f32),
        pltpu.VMEM((BLK, AUX), _f32),
        pltpu.VMEM((BLK, AUX), _f32),
        pltpu.VMEM((BLK, AUX), _f32),
        pltpu.VMEM((BLK // 8, 128), _f32),
        pltpu.VMEM((NB1 * SUB, CH), jnp.int32),
        pltpu.VMEM((NB2 * SUB, CH), jnp.int32),
        pltpu.SemaphoreType.DMA,
        pltpu.SemaphoreType.DMA,
        pltpu.SemaphoreType.DMA,
        pltpu.SemaphoreType.DMA,
    ],
)
def _stage_a(p_hbm, dst2d_hbm, zero_hbm, a_hbm,
             den_sh, p0, p1, d0, d1, sv, av, dstA, dstB,
             semp0, semp1, semg0, semg1):
    c = lax.axis_index("c")
    s = lax.axis_index("s")
    wid = s * NC + c
    row0 = s * ROWS_PT
    pltpu.sync_copy(zero_hbm.at[pl.ds(row0, ROWS_PT)],
                    den_sh.at[pl.ds(row0, ROWS_PT)])
    pltpu.sync_copy(dst2d_hbm.at[pl.ds(s * NB1 * SUB, NB1 * SUB)], dstA)
    pltpu.sync_copy(dst2d_hbm.at[pl.ds(wid * NB2 * SUB, NB2 * SUB)], dstB)
    plsc.subcore_barrier()

    pbufs = (p0, p1)
    dbufs = (d0, d1)
    psems = (semp0, semp1)
    gsems = (semg0, semg1)

    def p_src1(i):
        return p_hbm.at[pl.ds((s * NB1 + i) * (BLK // 8), BLK // 8)]

    pltpu.async_copy(p_src1(0), p0, semp0)
    pltpu.async_copy(p_src1(1), p1, semp1)

    def blk1(k, _):
        for b in (0, 1):
            i = 2 * k + b
            pb = pbufs[b]
            pltpu.make_async_copy(p_src1(i), pb, psems[b]).wait()

            def row(r):
                for q in range(8):
                    sv[r * 8 + q, :] = _leaky_exp(pb[r, pl.ds(q * L, L)])

            _vloop(BLK // 8, row, unroll=1)
            for u in range(SUB):
                pltpu.sync_copy(sv.at[pl.ds(u * CH, CH)],
                                den_sh.at[dstA.at[i * SUB + u]], add=True)

            @pl.when(i + 2 < NB1)
            def _():
                pltpu.async_copy(p_src1(i + 2), pb, psems[b])
        return 0

    lax.fori_loop(0, NB1 // 2, blk1, 0, unroll=False)
    plsc.subcore_barrier()

    def p_src2(i):
        return p_hbm.at[pl.ds((wid * NB2 + i) * (BLK // 8), BLK // 8)]

    def gathers(i, b):
        for u in range(SUB):
            pltpu.async_copy(den_sh.at[dstB.at[i * SUB + u]],
                             dbufs[b].at[pl.ds(u * CH, CH)], gsems[b])

    pltpu.async_copy(p_src2(0), p0, semp0)
    gathers(0, 0)
    pltpu.async_copy(p_src2(1), p1, semp1)
    gathers(1, 1)

    def blk2(k, _):
        for b in (0, 1):
            i = 2 * k + b
            pb = pbufs[b]
            db = dbufs[b]
            pltpu.make_async_copy(p_src2(i), pb, psems[b]).wait()
            for u in range(SUB):
                pltpu.make_async_copy(den_sh.at[dstB.at[i * SUB + u]],
                                      db.at[pl.ds(u * CH, CH)],
                                      gsems[b]).wait()

            def row(r):
                for q in range(8):
                    sl = pl.ds(q * L, L)
                    av[r, sl] = _leaky_exp(pb[r, sl]) / db[r * 8 + q, :]

            _vloop(BLK // 8, row, unroll=1)
            pltpu.sync_copy(
                av, a_hbm.at[pl.ds((wid * NB2 + i) * (BLK // 8), BLK // 8)])

            @pl.when(i + 2 < NB2)
            def _():
                pltpu.async_copy(p_src2(i + 2), pb, psems[b])
                gathers(i + 2, b)
        return 0

    lax.fori_loop(0, NB2 // 2, blk2, 0, unroll=False)


def _mm_body(a_ref, wlo_ref, whi_ref, blo_ref, bhi_ref, lo_ref, hi_ref):
    a = a_ref[...]
    lo_ref[...] = jnp.dot(a, wlo_ref[...],
                          preferred_element_type=_f32) + blo_ref[...]
    hi_ref[...] = jnp.dot(a, whi_ref[...],
                          preferred_element_type=_f32) + bhi_ref[...]


def _linear(a2d, wt, b):
    wexp = jnp.zeros((8, AUX, 8, HALF), _f32)
    wlo = wexp.at[jnp.arange(8), :, jnp.arange(8), :].set(wt[:, :HALF])
    whi = wexp.at[jnp.arange(8), :, jnp.arange(8), :].set(wt[:, HALF:])
    wlo = wlo.reshape(128, 8 * HALF)
    whi = whi.reshape(128, 8 * HALF)
    blo = jnp.tile(b[:HALF], 8)[None, :]
    bhi = jnp.tile(b[HALF:], 8)[None, :]
    MB = 128
    outs = pl.pallas_call(
        _mm_body,
        grid=(E_PAD // 8 // MB,),
        in_specs=[
            pl.BlockSpec((MB, 128), lambda i: (i, 0)),
            pl.BlockSpec((128, 8 * HALF), lambda i: (0, 0)),
            pl.BlockSpec((128, 8 * HALF), lambda i: (0, 0)),
            pl.BlockSpec((1, 8 * HALF), lambda i: (0, 0)),
            pl.BlockSpec((1, 8 * HALF), lambda i: (0, 0)),
        ],
        out_specs=[
            pl.BlockSpec((MB, 8 * HALF), lambda i: (i, 0)),
            pl.BlockSpec((MB, 8 * HALF), lambda i: (i, 0)),
        ],
        out_shape=[
            jax.ShapeDtypeStruct((E_PAD // 8, 8 * HALF), _f32),
            jax.ShapeDtypeStruct((E_PAD // 8, 8 * HALF), _f32),
        ],
    )(a2d, wlo, whi, blo, bhi)
    return outs


@functools.partial(
    pl.kernel,
    out_type=(
        jax.ShapeDtypeStruct((N_PAD, HALF), _f32),
        jax.ShapeDtypeStruct((N_PAD, HALF), _f32),
    ),
    mesh=_MESH,
    compiler_params=_TILED,
    scratch_types=[
        pltpu.VMEM_SHARED((N_PAD, HALF), _f32),
        pltpu.VMEM((CC, HALF), _f32),
        pltpu.VMEM((CC, HALF), _f32),
        pltpu.VMEM((CC // 8, 8 * HALF), _f32),
        pltpu.VMEM((CC // 8, 8 * HALF), _f32),
        pltpu.VMEM((CC, HALF), _f32),
        pltpu.VMEM((CC, HALF), _f32),
        pltpu.VMEM((CC,), jnp.int32),
        pltpu.VMEM((CC,), jnp.int32),
        pltpu.VMEM((CC,), jnp.int32),
        pltpu.VMEM((CC,), jnp.int32),
        pltpu.VMEM((CC,), jnp.int32),
        pltpu.VMEM((CC,), jnp.int32),
        pltpu.VMEM((CC,), jnp.int32),
        pltpu.VMEM((CC,), jnp.int32),
        pltpu.SemaphoreType.DMA,
        pltpu.SemaphoreType.DMA,
        pltpu.SemaphoreType.DMA,
        pltpu.SemaphoreType.DMA,
        pltpu.SemaphoreType.DMA,
        pltpu.SemaphoreType.DMA,
        pltpu.SemaphoreType.DMA,
        pltpu.SemaphoreType.DMA,
    ],
)
def _stage_c(srclo_hbm, srchi_hbm, alo_hbm, ahi_hbm, packed_hbm,
             zero_hbm, outlo_hbm, outhi_hbm,
             acc_sh, r0, r1, a0, a1, p0, p1, k0, k1,
             s0, s1, t0, t1, u0, u1,
             semr0, semr1, sema0, sema1, sems0, sems1, semk0, semk1):
    c = lax.axis_index("c")
    s = lax.axis_index("s")
    row0 = s * ROWS_PT
    pltpu.sync_copy(zero_hbm.at[pl.ds(row0, ROWS_PT)],
                    acc_sh.at[pl.ds(row0, ROWS_PT)])
    plsc.subcore_barrier()

    rbufs = (r0, r1)
    abufs = (a0, a1)
    pbufs = (p0, p1)
    kbufs = (k0, k1)
    sbufs = (s0, s1)
    tbufs = (t0, t1)
    ubufs = (u0, u1)
    rsems = (semr0, semr1)
    asems = (sema0, sema1)
    ssems = (sems0, sems1)
    ksems = (semk0, semk1)

    def k_src(i):
        return packed_hbm.at[pl.ds((s * NCC + i) * CC, CC)]

    def unpack_idx(i, b):
        for q in range(CC // L):
            sl = pl.ds(q * L, L)
            pk = kbufs[b][sl]
            sbufs[b][sl] = lax.shift_right_logical(pk, 14)
            tbufs[b][sl] = lax.bitwise_and(pk, PACK - 1)

    def run(tbl_hbm, a_half_hbm):
        def g_src(i, b):
            return tbl_hbm.at[sbufs[b]]

        def a_src(i):
            return a_half_hbm.at[pl.ds((s * NCC + i) * (CC // 8), CC // 8)]

        pltpu.async_copy(k_src(0), k0, semk0)
        pltpu.async_copy(k_src(1), k1, semk1)
        pltpu.make_async_copy(k_src(0), k0, semk0).wait()
        unpack_idx(0, 0)
        pltpu.async_copy(g_src(0, 0), r0, semr0)
        pltpu.async_copy(a_src(0), a0, sema0)
        pltpu.async_copy(k_src(2), k0, semk0)
        pltpu.make_async_copy(k_src(1), k1, semk1).wait()
        unpack_idx(1, 1)
        pltpu.async_copy(g_src(1, 1), r1, semr1)
        pltpu.async_copy(a_src(1), a1, sema1)
        pltpu.async_copy(k_src(3), k1, semk1)

        def chunk(k, _):
            for b in (0, 1):
                i = 2 * k + b
                rb = rbufs[b]
                ab = abufs[b]
                pb = pbufs[b]
                ub = ubufs[b]
                pltpu.make_async_copy(g_src(i, b), rb, rsems[b]).wait()
                pltpu.make_async_copy(a_src(i), ab, asems[b]).wait()

                @pl.when(i >= 2)
                def _():
                    pltpu.make_async_copy(pb, acc_sh.at[ub],
                                          ssems[b]).wait()

                for q in range(CC // L):
                    sl = pl.ds(q * L, L)
                    ub[sl] = tbufs[b][sl]

                def row(m):
                    for e8 in range(8):
                        j = m * 8 + e8
                        for q in range(HALF // L):
                            pb[j, pl.ds(q * L, L)] = (
                                rb[j, pl.ds(q * L, L)]
                                * ab[m, pl.ds(e8 * HALF + q * L, L)])

                _vloop(CC // 8, row, unroll=1)
                pltpu.async_copy(pb, acc_sh.at[ub], ssems[b], add=True)

                @pl.when(i + 2 < NCC)
                def _():
                    pltpu.make_async_copy(k_src(i + 2), kbufs[b],
                                          ksems[b]).wait()
                    unpack_idx(i + 2, b)
                    pltpu.async_copy(g_src(i + 2, b), rb, rsems[b])
                    pltpu.async_copy(a_src(i + 2), ab, asems[b])

                    @pl.when(i + 4 < NCC)
                    def _():
                        pltpu.async_copy(k_src(i + 4), kbufs[b], ksems[b])
            return 0

        lax.fori_loop(0, NCC // 2, chunk, 0, unroll=False)
        for b in (0, 1):
            pltpu.make_async_copy(pbufs[b], acc_sh.at[ubufs[b]],
                                  ssems[b]).wait()

    @pl.when(c == 0)
    def _():
        run(srclo_hbm, alo_hbm)

    @pl.when(c == 1)
    def _():
        run(srchi_hbm, ahi_hbm)

    plsc.subcore_barrier()

    @pl.when(c == 0)
    def _():
        pltpu.sync_copy(acc_sh.at[pl.ds(row0, ROWS_PT)],
                        outlo_hbm.at[pl.ds(row0, ROWS_PT)])

    @pl.when(c == 1)
    def _():
        pltpu.sync_copy(acc_sh.at[pl.ds(row0, ROWS_PT)],
                        outhi_hbm.at[pl.ds(row0, ROWS_PT)])


def kernel(src_ft, dst_ft, edge_p, edge_index, W_pos_w, W_pos_b):
    src = edge_index[0]
    dst = edge_index[1]
    pad = E_PAD - E
    p2d = jnp.pad(edge_p.reshape(E // 8, 128), ((0, pad // 8), (0, 0)))
    src_pad = jnp.pad(src, (0, pad))
    dst_pad = jnp.pad(dst, (0, pad), constant_values=N)
    dst2d = dst_pad.reshape(E_PAD // CH, CH)
    packed1d = src_pad * PACK + dst_pad
    zero_aux = jnp.zeros((N_PAD, AUX), _f32)
    zero_half = jnp.zeros((N_PAD, HALF), _f32)

    a2d = _stage_a(p2d, dst2d, zero_aux)
    a_lo, a_hi = _linear(a2d, W_pos_w.T, W_pos_b)
    out_lo, out_hi = _stage_c(src_ft[:, :HALF], src_ft[:, HALF:],
                              a_lo, a_hi, packed1d, zero_half)
    out = jnp.concatenate([out_lo[:N], out_hi[:N]], axis=1)
    return out[:, None, :]


# speedup vs baseline: 1.4838x; 1.0929x over previous
import functools

import jax
import jax.numpy as jnp
from jax import lax
from jax.experimental import pallas as pl
from jax.experimental.pallas import tpu as pltpu
from jax.experimental.pallas import tpu_sc as plsc

N = 10000
E = 160000
AUX = 16
OUT = 256
HALF = OUT // 2
NEG = 0.2

NC, NS, L = 2, 16, 16
NW = NC * NS
CH = 128
CC = 64
BLK = 512
SUB = BLK // CH
E_PAD = 163840
EPW = E_PAD // NW
EPT = E_PAD // NS
NB1 = EPT // BLK
NB2 = EPW // BLK
NCC = EPT // CC
PACK = 16384
N_PAD = 10112
ROWS_PT = N_PAD // NS

_MESH = plsc.VectorSubcoreMesh(core_axis_name="c", subcore_axis_name="s")
_f32 = jnp.float32
_bf16 = jnp.bfloat16
_NOTC = pltpu.CompilerParams(use_tc_tiling_on_sc=False,
                             needs_layout_passes=False)
_TILED = pltpu.CompilerParams(use_tc_tiling_on_sc=True,
                              needs_layout_passes=False)


def _leaky_exp(x):
    return jnp.exp(jnp.where(x >= 0, x, NEG * x))


def _vloop(n, body, unroll=8):
    def f(j, carry):
        body(j)
        return carry
    lax.fori_loop(0, n, f, 0, unroll=unroll)


@functools.partial(
    pl.kernel,
    out_type=jax.ShapeDtypeStruct((E_PAD // 8, 128), _f32),
    mesh=_MESH,
    compiler_params=_NOTC,
    scratch_types=[
        pltpu.VMEM_SHARED((N_PAD, AUX), _f32),
        pltpu.VMEM((BLK // 8, 128), _f32),
        pltpu.VMEM((BLK // 8, 128), _f32),
        pltpu.VMEM((BLK, AUX), _f32),
        pltpu.VMEM((BLK, AUX), _f32),
        pltpu.VMEM((BLK, AUX), _f32),
        pltpu.VMEM((BLK // 8, 128), _f32),
        pltpu.VMEM((NB1 * SUB, CH), jnp.int32),
        pltpu.VMEM((NB2 * SUB, CH), jnp.int32),
        pltpu.SemaphoreType.DMA,
        pltpu.SemaphoreType.DMA,
        pltpu.SemaphoreType.DMA,
        pltpu.SemaphoreType.DMA,
    ],
)
def _stage_a(p_hbm, dst2d_hbm, zero_hbm, a_hbm,
             den_sh, p0, p1, d0, d1, sv, av, dstA, dstB,
             semp0, semp1, semg0, semg1):
    c = lax.axis_index("c")
    s = lax.axis_index("s")
    wid = s * NC + c
    row0 = s * ROWS_PT
    pltpu.sync_copy(zero_hbm.at[pl.ds(row0, ROWS_PT)],
                    den_sh.at[pl.ds(row0, ROWS_PT)])
    pltpu.sync_copy(dst2d_hbm.at[pl.ds(s * NB1 * SUB, NB1 * SUB)], dstA)
    pltpu.sync_copy(dst2d_hbm.at[pl.ds(wid * NB2 * SUB, NB2 * SUB)], dstB)
    plsc.subcore_barrier()

    pbufs = (p0, p1)
    dbufs = (d0, d1)
    psems = (semp0, semp1)
    gsems = (semg0, semg1)

    def p_src1(i):
        return p_hbm.at[pl.ds((s * NB1 + i) * (BLK // 8), BLK // 8)]

    pltpu.async_copy(p_src1(0), p0, semp0)
    pltpu.async_copy(p_src1(1), p1, semp1)

    def blk1(k, _):
        for b in (0, 1):
            i = 2 * k + b
            pb = pbufs[b]
            pltpu.make_async_copy(p_src1(i), pb, psems[b]).wait()

            def row(r):
                for q in range(8):
                    sv[r * 8 + q, :] = _leaky_exp(pb[r, pl.ds(q * L, L)])

            _vloop(BLK // 8, row, unroll=1)
            for u in range(SUB):
                pltpu.sync_copy(sv.at[pl.ds(u * CH, CH)],
                                den_sh.at[dstA.at[i * SUB + u]], add=True)

            @pl.when(i + 2 < NB1)
            def _():
                pltpu.async_copy(p_src1(i + 2), pb, psems[b])
        return 0

    lax.fori_loop(0, NB1 // 2, blk1, 0, unroll=False)
    plsc.subcore_barrier()

    def p_src2(i):
        return p_hbm.at[pl.ds((wid * NB2 + i) * (BLK // 8), BLK // 8)]

    def gathers(i, b):
        for u in range(SUB):
            pltpu.async_copy(den_sh.at[dstB.at[i * SUB + u]],
                             dbufs[b].at[pl.ds(u * CH, CH)], gsems[b])

    pltpu.async_copy(p_src2(0), p0, semp0)
    gathers(0, 0)
    pltpu.async_copy(p_src2(1), p1, semp1)
    gathers(1, 1)

    def blk2(k, _):
        for b in (0, 1):
            i = 2 * k + b
            pb = pbufs[b]
            db = dbufs[b]
            pltpu.make_async_copy(p_src2(i), pb, psems[b]).wait()
            for u in range(SUB):
                pltpu.make_async_copy(den_sh.at[dstB.at[i * SUB + u]],
                                      db.at[pl.ds(u * CH, CH)],
                                      gsems[b]).wait()

            def row(r):
                for q in range(8):
                    sl = pl.ds(q * L, L)
                    av[r, sl] = _leaky_exp(pb[r, sl]) / db[r * 8 + q, :]

            _vloop(BLK // 8, row, unroll=1)
            pltpu.sync_copy(
                av, a_hbm.at[pl.ds((wid * NB2 + i) * (BLK // 8), BLK // 8)])

            @pl.when(i + 2 < NB2)
            def _():
                pltpu.async_copy(p_src2(i + 2), pb, psems[b])
                gathers(i + 2, b)
        return 0

    lax.fori_loop(0, NB2 // 2, blk2, 0, unroll=False)


def _mm_body(a_ref, wlo_ref, whi_ref, blo_ref, bhi_ref, lo_ref, hi_ref):
    a = a_ref[...]
    lo_ref[...] = jnp.dot(a, wlo_ref[...],
                          preferred_element_type=_f32) + blo_ref[...]
    hi_ref[...] = jnp.dot(a, whi_ref[...],
                          preferred_element_type=_f32) + bhi_ref[...]


def _linear(a2d, wt, b):
    wexp = jnp.zeros((8, AUX, 8, HALF), _f32)
    wlo = wexp.at[jnp.arange(8), :, jnp.arange(8), :].set(wt[:, :HALF])
    whi = wexp.at[jnp.arange(8), :, jnp.arange(8), :].set(wt[:, HALF:])
    wlo = wlo.reshape(128, 8 * HALF)
    whi = whi.reshape(128, 8 * HALF)
    blo = jnp.tile(b[:HALF], 8)[None, :]
    bhi = jnp.tile(b[HALF:], 8)[None, :]
    MB = 512
    outs = pl.pallas_call(
        _mm_body,
        grid=(E_PAD // 8 // MB,),
        in_specs=[
            pl.BlockSpec((MB, 128), lambda i: (i, 0)),
            pl.BlockSpec((128, 8 * HALF), lambda i: (0, 0)),
            pl.BlockSpec((128, 8 * HALF), lambda i: (0, 0)),
            pl.BlockSpec((1, 8 * HALF), lambda i: (0, 0)),
            pl.BlockSpec((1, 8 * HALF), lambda i: (0, 0)),
        ],
        out_specs=[
            pl.BlockSpec((MB, 8 * HALF), lambda i: (i, 0)),
            pl.BlockSpec((MB, 8 * HALF), lambda i: (i, 0)),
        ],
        out_shape=[
            jax.ShapeDtypeStruct((E_PAD // 8, 8 * HALF), _f32),
            jax.ShapeDtypeStruct((E_PAD // 8, 8 * HALF), _f32),
        ],
    )(a2d, wlo, whi, blo, bhi)
    return outs


@functools.partial(
    pl.kernel,
    out_type=(
        jax.ShapeDtypeStruct((N_PAD, HALF), _f32),
        jax.ShapeDtypeStruct((N_PAD, HALF), _f32),
    ),
    mesh=_MESH,
    compiler_params=_TILED,
    scratch_types=[
        pltpu.VMEM_SHARED((N_PAD, HALF), _f32),
        pltpu.VMEM((CC, HALF), _f32),
        pltpu.VMEM((CC, HALF), _f32),
        pltpu.VMEM((CC // 8, 8 * HALF), _f32),
        pltpu.VMEM((CC // 8, 8 * HALF), _f32),
        pltpu.VMEM((CC, HALF), _f32),
        pltpu.VMEM((CC, HALF), _f32),
        pltpu.VMEM((CC,), jnp.int32),
        pltpu.VMEM((CC,), jnp.int32),
        pltpu.VMEM((CC,), jnp.int32),
        pltpu.VMEM((CC,), jnp.int32),
        pltpu.VMEM((CC,), jnp.int32),
        pltpu.VMEM((CC,), jnp.int32),
        pltpu.VMEM((CC,), jnp.int32),
        pltpu.VMEM((CC,), jnp.int32),
        pltpu.SemaphoreType.DMA,
        pltpu.SemaphoreType.DMA,
        pltpu.SemaphoreType.DMA,
        pltpu.SemaphoreType.DMA,
        pltpu.SemaphoreType.DMA,
        pltpu.SemaphoreType.DMA,
        pltpu.SemaphoreType.DMA,
        pltpu.SemaphoreType.DMA,
    ],
)
def _stage_c(srclo_hbm, srchi_hbm, alo_hbm, ahi_hbm, packed_hbm,
             zero_hbm, outlo_hbm, outhi_hbm,
             acc_sh, r0, r1, a0, a1, p0, p1, k0, k1,
             s0, s1, t0, t1, u0, u1,
             semr0, semr1, sema0, sema1, sems0, sems1, semk0, semk1):
    c = lax.axis_index("c")
    s = lax.axis_index("s")
    row0 = s * ROWS_PT
    pltpu.sync_copy(zero_hbm.at[pl.ds(row0, ROWS_PT)],
                    acc_sh.at[pl.ds(row0, ROWS_PT)])
    plsc.subcore_barrier()

    rbufs = (r0, r1)
    abufs = (a0, a1)
    pbufs = (p0, p1)
    kbufs = (k0, k1)
    sbufs = (s0, s1)
    tbufs = (t0, t1)
    ubufs = (u0, u1)
    rsems = (semr0, semr1)
    asems = (sema0, sema1)
    ssems = (sems0, sems1)
    ksems = (semk0, semk1)

    def k_src(i):
        return packed_hbm.at[pl.ds((s * NCC + i) * CC, CC)]

    def unpack_idx(i, b):
        for q in range(CC // L):
            sl = pl.ds(q * L, L)
            pk = kbufs[b][sl]
            sbufs[b][sl] = lax.shift_right_logical(pk, 14)
            tbufs[b][sl] = lax.bitwise_and(pk, PACK - 1)

    def run(tbl_hbm, a_half_hbm):
        def g_src(i, b):
            return tbl_hbm.at[sbufs[b]]

        def a_src(i):
            return a_half_hbm.at[pl.ds((s * NCC + i) * (CC // 8), CC // 8)]

        pltpu.async_copy(k_src(0), k0, semk0)
        pltpu.async_copy(k_src(1), k1, semk1)
        pltpu.make_async_copy(k_src(0), k0, semk0).wait()
        unpack_idx(0, 0)
        pltpu.async_copy(g_src(0, 0), r0, semr0)
        pltpu.async_copy(a_src(0), a0, sema0)
        pltpu.async_copy(k_src(2), k0, semk0)
        pltpu.make_async_copy(k_src(1), k1, semk1).wait()
        unpack_idx(1, 1)
        pltpu.async_copy(g_src(1, 1), r1, semr1)
        pltpu.async_copy(a_src(1), a1, sema1)
        pltpu.async_copy(k_src(3), k1, semk1)

        def chunk(k, _):
            for b in (0, 1):
                i = 2 * k + b
                rb = rbufs[b]
                ab = abufs[b]
                pb = pbufs[b]
                ub = ubufs[b]
                pltpu.make_async_copy(g_src(i, b), rb, rsems[b]).wait()
                pltpu.make_async_copy(a_src(i), ab, asems[b]).wait()

                @pl.when(i >= 2)
                def _():
                    pltpu.make_async_copy(pb, acc_sh.at[ub],
                                          ssems[b]).wait()

                for q in range(CC // L):
                    sl = pl.ds(q * L, L)
                    ub[sl] = tbufs[b][sl]

                def row(m):
                    for e8 in range(8):
                        j = m * 8 + e8
                        for q in range(HALF // L):
                            pb[j, pl.ds(q * L, L)] = (
                                rb[j, pl.ds(q * L, L)]
                                * ab[m, pl.ds(e8 * HALF + q * L, L)])

                _vloop(CC // 8, row, unroll=1)
                pltpu.async_copy(pb, acc_sh.at[ub], ssems[b], add=True)

                @pl.when(i + 2 < NCC)
                def _():
                    pltpu.make_async_copy(k_src(i + 2), kbufs[b],
                                          ksems[b]).wait()
                    unpack_idx(i + 2, b)
                    pltpu.async_copy(g_src(i + 2, b), rb, rsems[b])
                    pltpu.async_copy(a_src(i + 2), ab, asems[b])

                    @pl.when(i + 4 < NCC)
                    def _():
                        pltpu.async_copy(k_src(i + 4), kbufs[b], ksems[b])
            return 0

        lax.fori_loop(0, NCC // 2, chunk, 0, unroll=False)
        for b in (0, 1):
            pltpu.make_async_copy(pbufs[b], acc_sh.at[ubufs[b]],
                                  ssems[b]).wait()

    @pl.when(c == 0)
    def _():
        run(srclo_hbm, alo_hbm)

    @pl.when(c == 1)
    def _():
        run(srchi_hbm, ahi_hbm)

    plsc.subcore_barrier()

    @pl.when(c == 0)
    def _():
        pltpu.sync_copy(acc_sh.at[pl.ds(row0, ROWS_PT)],
                        outlo_hbm.at[pl.ds(row0, ROWS_PT)])

    @pl.when(c == 1)
    def _():
        pltpu.sync_copy(acc_sh.at[pl.ds(row0, ROWS_PT)],
                        outhi_hbm.at[pl.ds(row0, ROWS_PT)])


def kernel(src_ft, dst_ft, edge_p, edge_index, W_pos_w, W_pos_b):
    src = edge_index[0]
    dst = edge_index[1]
    pad = E_PAD - E
    p2d = jnp.pad(edge_p.reshape(E // 8, 128), ((0, pad // 8), (0, 0)))
    src_pad = jnp.pad(src, (0, pad))
    dst_pad = jnp.pad(dst, (0, pad), constant_values=N)
    dst2d = dst_pad.reshape(E_PAD // CH, CH)
    packed1d = src_pad * PACK + dst_pad
    zero_aux = jnp.zeros((N_PAD, AUX), _f32)
    zero_half = jnp.zeros((N_PAD, HALF), _f32)

    a2d = _stage_a(p2d, dst2d, zero_aux)
    a_lo, a_hi = _linear(a2d, W_pos_w.T, W_pos_b)
    out_lo, out_hi = _stage_c(src_ft[:, :HALF], src_ft[:, HALF:],
                              a_lo, a_hi, packed1d, zero_half)
    out = jnp.concatenate([out_lo[:N], out_hi[:N]], axis=1)
    return out[:, None, :]
